# Initial kernel scaffold; baseline (speedup 1.0000x reference)
import jax
import jax.numpy as jnp
from jax.experimental import pallas as pl


def kernel(edge_index, features, W1, b1, W2, b2):
    raise NotImplementedError("write your pallas kernel here")


# speedup vs baseline: 2.1636x; 2.1636x over previous
import functools

import jax
import jax.numpy as jnp
from jax import lax
from jax.experimental import pallas as pl
from jax.experimental.pallas import tpu as pltpu
from jax.experimental.pallas import tpu_sc as plsc

_N = 10000
_E = 320000
_NW = 32
_NPW = 320
_NPAD = _NW * _NPW
_ACCR = _NPW + 8
_EPT = _E // _NW
_SCE = 2000
_FCAP = 256
_CAPH = 10240
_GB = 64
_LTOT = _NW * _NW * _CAPH


def _mesh():
  return plsc.VectorSubcoreMesh(core_axis_name="c", subcore_axis_name="s")


def _wid():
  return lax.axis_index("s") * 2 + lax.axis_index("c")


def _bucketize():
  @functools.partial(
      pl.kernel,
      mesh=_mesh(),
      out_type=(jax.ShapeDtypeStruct((_LTOT,), jnp.int32),
                jax.ShapeDtypeStruct((_NW * _NW,), jnp.int32)),
      scratch_types=[
          pltpu.VMEM((_SCE,), jnp.int32),
          pltpu.VMEM((_SCE,), jnp.int32),
          pltpu.VMEM((_NW * _FCAP,), jnp.int32),
          pltpu.VMEM((_NW,), jnp.int32),
          pltpu.SMEM((_NW,), jnp.int32),
          pltpu.SMEM((_NW,), jnp.int32),
      ],
  )
  def k(src_h, dst_h, lists_h, counts_h, srcb, dstb, bb, cntv, pcnt, fcnt):
    t = _wid()
    iot = lax.iota(jnp.int32, 16)
    padvec = (iot * 8 + t * 16) * 512 + (_NPW + (iot & 7))

    def init(b, _):
      pcnt[b] = 0
      fcnt[b] = 0
      return 0

    lax.fori_loop(0, _NW, init, 0)

    def chunk(sc, _):
      off = t * _EPT + sc * _SCE
      pltpu.sync_copy(src_h.at[pl.ds(off, _SCE)], srcb)
      pltpu.sync_copy(dst_h.at[pl.ds(off, _SCE)], dstb)

      def scan(j, _):
        sv = srcb[pl.ds(j * 16, 16)]
        dv = dstb[pl.ds(j * 16, 16)]
        bk = ((dv >> 6) * 52429) >> 18
        pk = sv * 512 + (dv - bk * _NPW)
        for l in range(16):
          b = bk[l]
          p = pk[l]
          pos = pcnt[b]
          wb = b * _FCAP + ((pos >> 4) << 4)
          w = bb[pl.ds(wb, 16)]
          bb[pl.ds(wb, 16)] = jnp.where(iot == (pos & 15), p, w)
          posn = pos + 1
          pcnt[b] = posn & (_FCAP - 1)

          @pl.when(posn == _FCAP)
          def _():
            nf = fcnt[b]
            pltpu.sync_copy(
                bb.at[pl.ds(b * _FCAP, _FCAP)],
                lists_h.at[pl.ds((t * _NW + b) * _CAPH + nf * _FCAP, _FCAP)])
            fcnt[b] = nf + 1
        return 0

      lax.fori_loop(0, _SCE // 16, scan, 0)
      return 0

    lax.fori_loop(0, _EPT // _SCE, chunk, 0)

    def fin(b, _):
      pos = pcnt[b]
      for w in range(_FCAP // 16):
        wb = b * _FCAP + w * 16
        cur = bb[pl.ds(wb, 16)]
        bb[pl.ds(wb, 16)] = jnp.where(iot + (w * 16) >= pos, padvec, cur)
      nf = fcnt[b]
      pltpu.sync_copy(
          bb.at[pl.ds(b * _FCAP, _FCAP)],
          lists_h.at[pl.ds((t * _NW + b) * _CAPH + nf * _FCAP, _FCAP)])
      stored = nf * _FCAP + (((pos + _GB - 1) >> 6) << 6)
      cb = (b >> 4) << 4
      cw = cntv[pl.ds(cb, 16)]
      cntv[pl.ds(cb, 16)] = jnp.where(iot == (b & 15), stored, cw)
      return 0

    lax.fori_loop(0, _NW, fin, 0)
    pltpu.sync_copy(cntv, counts_h.at[pl.ds(t * _NW, _NW)])

  return k


def _seg_max_128():
  @functools.partial(
      pl.kernel,
      mesh=_mesh(),
      out_type=jax.ShapeDtypeStruct((_NPAD, 128), jnp.float32),
      scratch_types=[
          pltpu.VMEM((_GB + 16,), jnp.int32),
          pltpu.VMEM((_GB,), jnp.int32),
          pltpu.VMEM((_GB, 128), jnp.float32),
          pltpu.VMEM((_ACCR, 128), jnp.float32),
          pltpu.VMEM((_NW * _NW + 16,), jnp.int32),
          pltpu.SemaphoreType.DMA,
      ],
  )
  def k(lists_h, counts_h, feat_h, out_h, lbuf, gidx, rows, acc, cbuf, sem):
    w = _wid()
    neg = jnp.full((16,), -jnp.inf, dtype=jnp.float32)

    def init(r, _):
      for kk in range(8):
        acc[r, pl.ds(kk * 16, 16)] = neg
      return 0

    lax.fori_loop(0, _ACCR, init, 0)
    pltpu.sync_copy(counts_h, cbuf.at[pl.ds(0, _NW * _NW)])

    def per_t(t, _):
      ct = cbuf[pl.ds(t * _NW + w, 16)][0]
      base0 = (t * _NW + w) * _CAPH

      def per_b(b2, _):
        pltpu.sync_copy(lists_h.at[pl.ds(base0 + b2 * _GB, _GB)],
                        lbuf.at[pl.ds(0, _GB)])
        for u in range(_GB // 16):
          gidx[pl.ds(u * 16, 16)] = lbuf[pl.ds(u * 16, 16)] >> 9
        pltpu.async_copy(feat_h.at[gidx], rows, sem).wait()

        def per_e(e, _):
          pk = lbuf[pl.ds(e, 16)][0]
          dl = pk & 511
          for kk in range(8):
            sl = pl.ds(kk * 16, 16)
            acc[dl, sl] = jnp.maximum(acc[dl, sl], rows[e, sl])
          return 0

        lax.fori_loop(0, _GB, per_e, 0)
        return 0

      lax.fori_loop(0, ct >> 6, per_b, 0)
      return 0

    lax.fori_loop(0, _NW, per_t, 0)
    pltpu.sync_copy(acc.at[pl.ds(0, _NPW)], out_h.at[pl.ds(w * _NPW, _NPW)])

  return k


def _seg_max_16():
  @functools.partial(
      pl.kernel,
      mesh=_mesh(),
      out_type=jax.ShapeDtypeStruct((_NPAD * 16,), jnp.float32),
      scratch_types=[
          pltpu.VMEM((_GB + 16,), jnp.int32),
          pltpu.VMEM((_GB,), jnp.int32),
          pltpu.VMEM((_GB, 128), jnp.float32),
          pltpu.VMEM((_ACCR * 16,), jnp.float32),
          pltpu.VMEM((_NW * _NW + 16,), jnp.int32),
          pltpu.SemaphoreType.DMA,
      ],
  )
  def k(lists_h, counts_h, hpk_h, out_h, lbuf, gidx, rows, acc, cbuf, sem):
    w = _wid()
    neg = jnp.full((16,), -jnp.inf, dtype=jnp.float32)

    def init(r, _):
      acc[pl.ds(r * 16, 16)] = neg
      return 0

    lax.fori_loop(0, _ACCR, init, 0)
    pltpu.sync_copy(counts_h, cbuf.at[pl.ds(0, _NW * _NW)])

    def per_t(t, _):
      ct = cbuf[pl.ds(t * _NW + w, 16)][0]
      base0 = (t * _NW + w) * _CAPH

      def per_b(b2, _):
        pltpu.sync_copy(lists_h.at[pl.ds(base0 + b2 * _GB, _GB)],
                        lbuf.at[pl.ds(0, _GB)])
        for u in range(_GB // 16):
          gidx[pl.ds(u * 16, 16)] = lbuf[pl.ds(u * 16, 16)] >> 12
        pltpu.async_copy(hpk_h.at[gidx], rows, sem).wait()

        def per_e(e, _):
          pk = lbuf[pl.ds(e, 16)][0]
          dl = pk & 511
          s = pk >> 9
          sub = (s & 7) * 16
          val = rows[e, pl.ds(sub, 16)]
          ab = pl.ds(dl * 16, 16)
          acc[ab] = jnp.maximum(acc[ab], val)
          return 0

        lax.fori_loop(0, _GB, per_e, 0)
        return 0

      lax.fori_loop(0, ct >> 6, per_b, 0)
      return 0

    lax.fori_loop(0, _NW, per_t, 0)
    pltpu.sync_copy(acc.at[pl.ds(0, _NPW * 16)],
                    out_h.at[pl.ds(w * _NPW * 16, _NPW * 16)])

  return k


def _dense1(x, agg, W1, b1):
  def body(x_ref, a_ref, w_ref, b_ref, o_ref):
    a = a_ref[...]
    a = jnp.where(a == -jnp.inf, 0.0, a)
    rst = x_ref[...] + a
    h = jnp.dot(rst, w_ref[...], preferred_element_type=jnp.float32)
    o_ref[...] = jnp.maximum(h + b_ref[...], 0.0)

  br = 2000
  return pl.pallas_call(
      body,
      grid=(_N // br,),
      in_specs=[
          pl.BlockSpec((br, 128), lambda i: (i, 0)),
          pl.BlockSpec((br, 128), lambda i: (i, 0)),
          pl.BlockSpec((128, 16), lambda i: (0, 0)),
          pl.BlockSpec((1, 16), lambda i: (0, 0)),
      ],
      out_specs=pl.BlockSpec((br, 16), lambda i: (i, 0)),
      out_shape=jax.ShapeDtypeStruct((_N, 16), jnp.float32),
  )(x, agg, W1, b1.reshape(1, 16))


def _dense2(h, agg, W2, b2):
  def body(h_ref, a_ref, w_ref, b_ref, o_ref):
    a = a_ref[...]
    a = jnp.where(a == -jnp.inf, 0.0, a)
    rst = h_ref[...] + a
    z = jnp.dot(rst, w_ref[...], preferred_element_type=jnp.float32)
    z = z + b_ref[...]
    m = jnp.max(z, axis=-1, keepdims=True)
    zm = z - m
    o_ref[...] = zm - jnp.log(jnp.sum(jnp.exp(zm), axis=-1, keepdims=True))

  br = 2000
  return pl.pallas_call(
      body,
      grid=(_N // br,),
      in_specs=[
          pl.BlockSpec((br, 16), lambda i: (i, 0)),
          pl.BlockSpec((br, 16), lambda i: (i, 0)),
          pl.BlockSpec((16, 7), lambda i: (0, 0)),
          pl.BlockSpec((1, 7), lambda i: (0, 0)),
      ],
      out_specs=pl.BlockSpec((br, 7), lambda i: (i, 0)),
      out_shape=jax.ShapeDtypeStruct((_N, 7), jnp.float32),
  )(h, agg, W2, b2.reshape(1, 7))


def kernel(edge_index, features, W1, b1, W2, b2):
  src = edge_index[0]
  dst = edge_index[1]
  lists, counts = _bucketize()(src, dst)
  agg1 = _seg_max_128()(lists, counts, features)[:_N]
  h = _dense1(features, agg1, W1, b1)
  hpk = jnp.concatenate(
      [h, jnp.zeros((_NPAD - _N, 16), jnp.float32)]).reshape(_NPAD // 8, 128)
  agg2 = _seg_max_16()(lists, counts, hpk).reshape(_NPAD, 16)[:_N]
  return _dense2(h, agg2, W2, b2)


# speedup vs baseline: 2.4605x; 1.1372x over previous
import functools

import jax
import jax.numpy as jnp
from jax import lax
from jax.experimental import pallas as pl
from jax.experimental.pallas import tpu as pltpu
from jax.experimental.pallas import tpu_sc as plsc

_N = 10000
_E = 320000
_NW = 32
_NPW = 320
_NPAD = _NW * _NPW
_ACCR = _NPW + 8
_EPT = _E // _NW
_SCE = 2000
_FCAP = 256
_CAPH = 10240
_GB = 128
_LTOT = _NW * _NW * _CAPH


def _mesh():
  return plsc.VectorSubcoreMesh(core_axis_name="c", subcore_axis_name="s")


def _wid():
  return lax.axis_index("s") * 2 + lax.axis_index("c")


def _bucketize():
  @functools.partial(
      pl.kernel,
      mesh=_mesh(),
      out_type=(jax.ShapeDtypeStruct((_LTOT,), jnp.int32),
                jax.ShapeDtypeStruct((_NW * _NW,), jnp.int32)),
      scratch_types=[
          pltpu.VMEM((_SCE,), jnp.int32),
          pltpu.VMEM((_SCE,), jnp.int32),
          pltpu.VMEM((_NW * _FCAP,), jnp.int32),
          pltpu.VMEM((_NW,), jnp.int32),
          pltpu.SMEM((_NW,), jnp.int32),
          pltpu.SMEM((_NW,), jnp.int32),
      ],
  )
  def k(src_h, dst_h, lists_h, counts_h, srcb, dstb, bb, cntv, pcnt, fcnt):
    t = _wid()
    iot = lax.iota(jnp.int32, 16)
    padvec = (iot * 8 + t * 16) * 512 + (_NPW + (iot & 7))

    def init(b, _):
      pcnt[b] = 0
      fcnt[b] = 0
      return 0

    lax.fori_loop(0, _NW, init, 0)

    def chunk(sc, _):
      off = t * _EPT + sc * _SCE
      pltpu.sync_copy(src_h.at[pl.ds(off, _SCE)], srcb)
      pltpu.sync_copy(dst_h.at[pl.ds(off, _SCE)], dstb)

      def scan(j, _):
        sv = srcb[pl.ds(j * 16, 16)]
        dv = dstb[pl.ds(j * 16, 16)]
        bk = ((dv >> 6) * 52429) >> 18
        pk = sv * 512 + (dv - bk * _NPW)
        for l in range(16):
          b = bk[l]
          p = pk[l]
          pos = pcnt[b]
          wb = b * _FCAP + ((pos >> 4) << 4)
          w = bb[pl.ds(wb, 16)]
          bb[pl.ds(wb, 16)] = jnp.where(iot == (pos & 15), p, w)
          posn = pos + 1
          pcnt[b] = posn & (_FCAP - 1)

          @pl.when(posn == _FCAP)
          def _():
            nf = fcnt[b]
            pltpu.sync_copy(
                bb.at[pl.ds(b * _FCAP, _FCAP)],
                lists_h.at[pl.ds((t * _NW + b) * _CAPH + nf * _FCAP, _FCAP)])
            fcnt[b] = nf + 1
        return 0

      lax.fori_loop(0, _SCE // 16, scan, 0)
      return 0

    lax.fori_loop(0, _EPT // _SCE, chunk, 0)

    def fin(b, _):
      pos = pcnt[b]
      for w in range(_FCAP // 16):
        wb = b * _FCAP + w * 16
        cur = bb[pl.ds(wb, 16)]
        bb[pl.ds(wb, 16)] = jnp.where(iot + (w * 16) >= pos, padvec, cur)
      nf = fcnt[b]
      pltpu.sync_copy(
          bb.at[pl.ds(b * _FCAP, _FCAP)],
          lists_h.at[pl.ds((t * _NW + b) * _CAPH + nf * _FCAP, _FCAP)])
      stored = nf * _FCAP + (((pos + _GB - 1) >> 7) << 7)
      cb = (b >> 4) << 4
      cw = cntv[pl.ds(cb, 16)]
      cntv[pl.ds(cb, 16)] = jnp.where(iot == (b & 15), stored, cw)
      return 0

    lax.fori_loop(0, _NW, fin, 0)
    pltpu.sync_copy(cntv, counts_h.at[pl.ds(t * _NW, _NW)])

  return k


def _seg_max_128():
  @functools.partial(
      pl.kernel,
      mesh=_mesh(),
      out_type=jax.ShapeDtypeStruct((_NPAD, 128), jnp.float32),
      scratch_types=[
          pltpu.VMEM((_CAPH + 16,), jnp.int32),
          pltpu.VMEM((_GB,), jnp.int32),
          pltpu.VMEM((_GB,), jnp.int32),
          pltpu.VMEM((_GB, 128), jnp.float32),
          pltpu.VMEM((_GB, 128), jnp.float32),
          pltpu.VMEM((_ACCR, 128), jnp.float32),
          pltpu.VMEM((_NW * _NW + 16,), jnp.int32),
          pltpu.SemaphoreType.DMA,
          pltpu.SemaphoreType.DMA,
      ],
  )
  def k(lists_h, counts_h, feat_h, out_h, llbuf, gidxA, gidxB, rowsA, rowsB,
        acc, cbuf, semA, semB):
    w = _wid()
    neg = jnp.full((16,), -jnp.inf, dtype=jnp.float32)

    def init(r, _):
      for kk in range(8):
        acc[r, pl.ds(kk * 16, 16)] = neg
      return 0

    lax.fori_loop(0, _ACCR, init, 0)
    pltpu.sync_copy(counts_h, cbuf.at[pl.ds(0, _NW * _NW)])

    def start(j, gidx, rows, sem):
      lbase = j * _GB
      for u in range(_GB // 16):
        gidx[pl.ds(u * 16, 16)] = llbuf[pl.ds(lbase + u * 16, 16)] >> 9
      return pltpu.async_copy(feat_h.at[gidx], rows, sem)

    def accum(j, rows):
      lbase = j * _GB

      def per_eo(eo, _):
        for sub in range(4):
          e = eo * 4 + sub
          pk = llbuf[pl.ds(lbase + e, 16)][0]
          dl = pk & 511
          for kk in range(8):
            sl = pl.ds(kk * 16, 16)
            acc[dl, sl] = jnp.maximum(acc[dl, sl], rows[e, sl])
        return 0

      lax.fori_loop(0, _GB // 4, per_eo, 0)

    def per_t(t, _):
      ct = cbuf[pl.ds(t * _NW + w, 16)][0]
      nb = ct >> 7
      base0 = (t * _NW + w) * _CAPH

      @pl.when(nb > 0)
      def _():
        def cp(cc, _):
          pltpu.sync_copy(lists_h.at[pl.ds(base0 + cc * 256, 256)],
                          llbuf.at[pl.ds(cc * 256, 256)])
          return 0

        lax.fori_loop(0, (ct + 255) >> 8, cp, 0)
        start(0, gidxA, rowsA, semA).wait()
        accum(0, rowsA)

        def ring(jj, _):
          j1 = jnp.minimum(jj * 2 + 1, nb - 1)
          j2 = jnp.minimum(jj * 2 + 2, nb - 1)
          cpB = start(j1, gidxB, rowsB, semB)
          cpA = start(j2, gidxA, rowsA, semA)
          cpB.wait()
          accum(j1, rowsB)
          cpA.wait()
          accum(j2, rowsA)
          return 0

        lax.fori_loop(0, nb >> 1, ring, 0)
      return 0

    lax.fori_loop(0, _NW, per_t, 0)
    pltpu.sync_copy(acc.at[pl.ds(0, _NPW)], out_h.at[pl.ds(w * _NPW, _NPW)])

  return k


def _seg_max_16():
  @functools.partial(
      pl.kernel,
      mesh=_mesh(),
      out_type=jax.ShapeDtypeStruct((_NPAD * 16,), jnp.float32),
      scratch_types=[
          pltpu.VMEM((_CAPH + 16,), jnp.int32),
          pltpu.VMEM((_GB,), jnp.int32),
          pltpu.VMEM((_GB,), jnp.int32),
          pltpu.VMEM((_GB, 128), jnp.float32),
          pltpu.VMEM((_GB, 128), jnp.float32),
          pltpu.VMEM((_ACCR * 16,), jnp.float32),
          pltpu.VMEM((_NW * _NW + 16,), jnp.int32),
          pltpu.SemaphoreType.DMA,
          pltpu.SemaphoreType.DMA,
      ],
  )
  def k(lists_h, counts_h, hpk_h, out_h, llbuf, gidxA, gidxB, rowsA, rowsB,
        acc, cbuf, semA, semB):
    w = _wid()
    neg = jnp.full((16,), -jnp.inf, dtype=jnp.float32)

    def init(r, _):
      acc[pl.ds(r * 16, 16)] = neg
      return 0

    lax.fori_loop(0, _ACCR, init, 0)
    pltpu.sync_copy(counts_h, cbuf.at[pl.ds(0, _NW * _NW)])

    def start(j, gidx, rows, sem):
      lbase = j * _GB
      for u in range(_GB // 16):
        gidx[pl.ds(u * 16, 16)] = llbuf[pl.ds(lbase + u * 16, 16)] >> 12
      return pltpu.async_copy(hpk_h.at[gidx], rows, sem)

    def accum(j, rows):
      lbase = j * _GB

      def per_eo(eo, _):
        for sub in range(4):
          e = eo * 4 + sub
          pk = llbuf[pl.ds(lbase + e, 16)][0]
          dl = pk & 511
          s = pk >> 9
          sub_off = (s & 7) * 16
          val = rows[e, pl.ds(sub_off, 16)]
          ab = pl.ds(dl * 16, 16)
          acc[ab] = jnp.maximum(acc[ab], val)
        return 0

      lax.fori_loop(0, _GB // 4, per_eo, 0)

    def per_t(t, _):
      ct = cbuf[pl.ds(t * _NW + w, 16)][0]
      nb = ct >> 7
      base0 = (t * _NW + w) * _CAPH

      @pl.when(nb > 0)
      def _():
        def cp(cc, _):
          pltpu.sync_copy(lists_h.at[pl.ds(base0 + cc * 256, 256)],
                          llbuf.at[pl.ds(cc * 256, 256)])
          return 0

        lax.fori_loop(0, (ct + 255) >> 8, cp, 0)
        start(0, gidxA, rowsA, semA).wait()
        accum(0, rowsA)

        def ring(jj, _):
          j1 = jnp.minimum(jj * 2 + 1, nb - 1)
          j2 = jnp.minimum(jj * 2 + 2, nb - 1)
          cpB = start(j1, gidxB, rowsB, semB)
          cpA = start(j2, gidxA, rowsA, semA)
          cpB.wait()
          accum(j1, rowsB)
          cpA.wait()
          accum(j2, rowsA)
          return 0

        lax.fori_loop(0, nb >> 1, ring, 0)
      return 0

    lax.fori_loop(0, _NW, per_t, 0)
    pltpu.sync_copy(acc.at[pl.ds(0, _NPW * 16)],
                    out_h.at[pl.ds(w * _NPW * 16, _NPW * 16)])

  return k


def _dense1(x, agg, W1, b1):
  def body(x_ref, a_ref, w_ref, b_ref, o_ref):
    a = a_ref[...]
    a = jnp.where(a == -jnp.inf, 0.0, a)
    rst = x_ref[...] + a
    h = jnp.dot(rst, w_ref[...], preferred_element_type=jnp.float32)
    o_ref[...] = jnp.maximum(h + b_ref[...], 0.0)

  br = 2000
  return pl.pallas_call(
      body,
      grid=(_N // br,),
      in_specs=[
          pl.BlockSpec((br, 128), lambda i: (i, 0)),
          pl.BlockSpec((br, 128), lambda i: (i, 0)),
          pl.BlockSpec((128, 16), lambda i: (0, 0)),
          pl.BlockSpec((1, 16), lambda i: (0, 0)),
      ],
      out_specs=pl.BlockSpec((br, 16), lambda i: (i, 0)),
      out_shape=jax.ShapeDtypeStruct((_N, 16), jnp.float32),
  )(x, agg, W1, b1.reshape(1, 16))


def _dense2(h, agg, W2, b2):
  def body(h_ref, a_ref, w_ref, b_ref, o_ref):
    a = a_ref[...]
    a = jnp.where(a == -jnp.inf, 0.0, a)
    rst = h_ref[...] + a
    z = jnp.dot(rst, w_ref[...], preferred_element_type=jnp.float32)
    z = z + b_ref[...]
    m = jnp.max(z, axis=-1, keepdims=True)
    zm = z - m
    o_ref[...] = zm - jnp.log(jnp.sum(jnp.exp(zm), axis=-1, keepdims=True))

  br = 2000
  return pl.pallas_call(
      body,
      grid=(_N // br,),
      in_specs=[
          pl.BlockSpec((br, 16), lambda i: (i, 0)),
          pl.BlockSpec((br, 16), lambda i: (i, 0)),
          pl.BlockSpec((16, 7), lambda i: (0, 0)),
          pl.BlockSpec((1, 7), lambda i: (0, 0)),
      ],
      out_specs=pl.BlockSpec((br, 7), lambda i: (i, 0)),
      out_shape=jax.ShapeDtypeStruct((_N, 7), jnp.float32),
  )(h, agg, W2, b2.reshape(1, 7))


def kernel(edge_index, features, W1, b1, W2, b2):
  src = edge_index[0]
  dst = edge_index[1]
  lists, counts = _bucketize()(src, dst)
  agg1 = _seg_max_128()(lists, counts, features)[:_N]
  h = _dense1(features, agg1, W1, b1)
  hpk = jnp.concatenate(
      [h, jnp.zeros((_NPAD - _N, 16), jnp.float32)]).reshape(_NPAD // 8, 128)
  agg2 = _seg_max_16()(lists, counts, hpk).reshape(_NPAD, 16)[:_N]
  return _dense2(h, agg2, W2, b2)


# speedup vs baseline: 3.6975x; 1.5027x over previous
import functools

import jax
import jax.numpy as jnp
from jax import lax
from jax.experimental import pallas as pl
from jax.experimental.pallas import tpu as pltpu
from jax.experimental.pallas import tpu_sc as plsc

_N = 10000
_E = 320000
_NW = 32
_NPW = 320
_NPAD = _NW * _NPW
_ACCR = _NPW + 8
_EPT = _E // _NW
_SCE = 2000
_FCAP = 256
_CAPH = 10240
_GB = 128
_LTOT = _NW * _NW * _CAPH
_BW = 32
_ABLK = 10752


def _mesh():
  return plsc.VectorSubcoreMesh(core_axis_name="c", subcore_axis_name="s")


def _wid():
  return lax.axis_index("s") * 2 + lax.axis_index("c")


def _bucketize():
  @functools.partial(
      pl.kernel,
      mesh=_mesh(),
      out_type=(jax.ShapeDtypeStruct((_LTOT,), jnp.int32),
                jax.ShapeDtypeStruct((_NW * _NW,), jnp.int32)),
      scratch_types=[
          pltpu.VMEM((_SCE,), jnp.int32),
          pltpu.VMEM((_SCE,), jnp.int32),
          pltpu.VMEM((_NW * _FCAP,), jnp.int32),
          pltpu.VMEM((_NW,), jnp.int32),
          pltpu.SMEM((_NW,), jnp.int32),
          pltpu.SMEM((_NW,), jnp.int32),
      ],
  )
  def k(src_h, dst_h, lists_h, counts_h, srcb, dstb, bb, cntv, pcnt, fcnt):
    t = _wid()
    iot = lax.iota(jnp.int32, 16)
    padvec = (iot * 8 + t * 16) * 512 + (_NPW + (iot & 7))

    def init(b, _):
      pcnt[b] = 0
      fcnt[b] = 0
      return 0

    lax.fori_loop(0, _NW, init, 0)

    def chunk(sc, _):
      off = t * _EPT + sc * _SCE
      pltpu.sync_copy(src_h.at[pl.ds(off, _SCE)], srcb)
      pltpu.sync_copy(dst_h.at[pl.ds(off, _SCE)], dstb)

      def scan(j, _):
        sv = srcb[pl.ds(j * 16, 16)]
        dv = dstb[pl.ds(j * 16, 16)]
        bk = ((dv >> 6) * 52429) >> 18
        pk = sv * 512 + (dv - bk * _NPW)
        for l in range(16):
          b = bk[l]
          p = pk[l]
          pos = pcnt[b]
          wb = b * _FCAP + ((pos >> 4) << 4)
          w = bb[pl.ds(wb, 16)]
          bb[pl.ds(wb, 16)] = jnp.where(iot == (pos & 15), p, w)
          posn = pos + 1
          pcnt[b] = posn & (_FCAP - 1)

          @pl.when(posn == _FCAP)
          def _():
            nf = fcnt[b]
            pltpu.sync_copy(
                bb.at[pl.ds(b * _FCAP, _FCAP)],
                lists_h.at[pl.ds((t * _NW + b) * _CAPH + nf * _FCAP, _FCAP)])
            fcnt[b] = nf + 1
        return 0

      lax.fori_loop(0, _SCE // 16, scan, 0)
      return 0

    lax.fori_loop(0, _EPT // _SCE, chunk, 0)

    def fin(b, _):
      pos = pcnt[b]
      for w in range(_FCAP // 16):
        wb = b * _FCAP + w * 16
        cur = bb[pl.ds(wb, 16)]
        bb[pl.ds(wb, 16)] = jnp.where(iot + (w * 16) >= pos, padvec, cur)
      nf = fcnt[b]
      pltpu.sync_copy(
          bb.at[pl.ds(b * _FCAP, _FCAP)],
          lists_h.at[pl.ds((t * _NW + b) * _CAPH + nf * _FCAP, _FCAP)])
      stored = nf * _FCAP + pos
      cb = (b >> 4) << 4
      cw = cntv[pl.ds(cb, 16)]
      cntv[pl.ds(cb, 16)] = jnp.where(iot == (b & 15), stored, cw)
      return 0

    lax.fori_loop(0, _NW, fin, 0)
    pltpu.sync_copy(cntv, counts_h.at[pl.ds(t * _NW, _NW)])

  return k


def _node_blocks():
  @functools.partial(
      pl.kernel,
      mesh=_mesh(),
      out_type=(jax.ShapeDtypeStruct((_NW * _ABLK * _BW,), jnp.int32),
                jax.ShapeDtypeStruct((_NW * _ABLK,), jnp.int32),
                jax.ShapeDtypeStruct((_NW * 16,), jnp.int32)),
      scratch_types=[
          pltpu.VMEM((_CAPH + 16,), jnp.int32),
          pltpu.VMEM((_ACCR * _BW,), jnp.int32),
          pltpu.VMEM((_ABLK,), jnp.int32),
          pltpu.VMEM((16,), jnp.int32),
          pltpu.VMEM((_NW * _NW + 16,), jnp.int32),
          pltpu.SMEM((_ACCR,), jnp.int32),
          pltpu.SMEM((_ACCR,), jnp.int32),
      ],
  )
  def k(lists_h, counts_h, blocks_h, blknode_h, counts2_h, llbuf, buf2,
        bnbuf, cv, cbuf, pcnt2, lastsrc):
    w = _wid()
    iot = lax.iota(jnp.int32, 16)
    arena = w * _ABLK * _BW

    def init(i, _):
      pcnt2[i] = 0
      return 0

    lax.fori_loop(0, _ACCR, init, 0)
    pltpu.sync_copy(counts_h, cbuf.at[pl.ds(0, _NW * _NW)])

    def append(srcv, dlv, l, na):
      src = srcv[l]
      dl = dlv[l]
      valid = dl < _NPW
      pos = pcnt2[dl]
      wb = (dl << 5) + ((pos >> 4) << 4)
      wold = buf2[pl.ds(wb, 16)]
      lane = jnp.where(valid, pos & 15, 16)
      buf2[pl.ds(wb, 16)] = jnp.where(iot == lane, src, wold)
      posn = pos + jnp.where(valid, 1, 0)
      pcnt2[dl] = posn & (_BW - 1)
      lastsrc[dl] = src
      flushed = posn == _BW

      @pl.when(flushed)
      def _():
        pltpu.sync_copy(buf2.at[pl.ds(dl * _BW, _BW)],
                        blocks_h.at[pl.ds(arena + na * _BW, _BW)])
        nwb = (na >> 4) << 4
        bw = bnbuf[pl.ds(nwb, 16)]
        bnbuf[pl.ds(nwb, 16)] = jnp.where(iot == (na & 15), dl, bw)

      return na + jnp.where(flushed, 1, 0)

    def per_t(t, na):
      rc = cbuf[pl.ds(t * _NW + w, 16)][0]
      base0 = (t * _NW + w) * _CAPH

      def cp(cc, _):
        pltpu.sync_copy(lists_h.at[pl.ds(base0 + cc * 256, 256)],
                        llbuf.at[pl.ds(cc * 256, 256)])
        return 0

      lax.fori_loop(0, (rc + 255) >> 8, cp, 0)

      def scanv(j, na2):
        pkv = llbuf[pl.ds(j * 16, 16)]
        srcv = pkv >> 9
        dlv = pkv & 511
        for l in range(16):
          na2 = append(srcv, dlv, l, na2)
        return na2

      return lax.fori_loop(0, (rc + 15) >> 4, scanv, na)

    na = lax.fori_loop(0, _NW, per_t, jnp.int32(0))

    def fin(dl, na2):
      pos = pcnt2[dl]
      any_ = pos > 0

      @pl.when(any_)
      def _():
        lsrc = lastsrc[dl]
        for wv in range(2):
          wb = (dl << 5) + wv * 16
          cur = buf2[pl.ds(wb, 16)]
          buf2[pl.ds(wb, 16)] = jnp.where(iot + wv * 16 >= pos, lsrc, cur)
        pltpu.sync_copy(buf2.at[pl.ds(dl * _BW, _BW)],
                        blocks_h.at[pl.ds(arena + na2 * _BW, _BW)])
        nwb = (na2 >> 4) << 4
        bw = bnbuf[pl.ds(nwb, 16)]
        bnbuf[pl.ds(nwb, 16)] = jnp.where(iot == (na2 & 15), dl, bw)

      return na2 + jnp.where(any_, 1, 0)

    na = lax.fori_loop(0, _ACCR, fin, na)

    padsrc = iot * 8 + w * 16
    npad = (-na) & 3
    for i in range(3):
      @pl.when(i < npad)
      def _():
        for wv in range(2):
          buf2[pl.ds((_NPW << 5) + wv * 16, 16)] = padsrc
        nb2 = na + i
        pltpu.sync_copy(buf2.at[pl.ds(_NPW << 5, _BW)],
                        blocks_h.at[pl.ds(arena + nb2 * _BW, _BW)])
        nwb = (nb2 >> 4) << 4
        bw = bnbuf[pl.ds(nwb, 16)]
        bnbuf[pl.ds(nwb, 16)] = jnp.where(iot == (nb2 & 15), _NPW, bw)

    na = na + npad
    cv[pl.ds(0, 16)] = jnp.where(iot == 0, na, 0)
    pltpu.sync_copy(cv, counts2_h.at[pl.ds(w * 16, 16)])

    def cpo(cc, _):
      pltpu.sync_copy(bnbuf.at[pl.ds(cc * 256, 256)],
                      blknode_h.at[pl.ds(w * _ABLK + cc * 256, 256)])
      return 0

    lax.fori_loop(0, (na + 255) >> 8, cpo, 0)

  return k


def _seg_max_128():
  @functools.partial(
      pl.kernel,
      mesh=_mesh(),
      out_type=jax.ShapeDtypeStruct((_NPAD, 128), jnp.float32),
      scratch_types=[
          pltpu.VMEM((_ABLK + 16,), jnp.int32),
          pltpu.VMEM((_GB,), jnp.int32),
          pltpu.VMEM((_GB,), jnp.int32),
          pltpu.VMEM((_GB, 128), jnp.float32),
          pltpu.VMEM((_GB, 128), jnp.float32),
          pltpu.VMEM((_ACCR, 128), jnp.float32),
          pltpu.VMEM((16 * _NW + 16,), jnp.int32),
          pltpu.SemaphoreType.DMA,
          pltpu.SemaphoreType.DMA,
          pltpu.SemaphoreType.DMA,
          pltpu.SemaphoreType.DMA,
      ],
  )
  def k(blocks_h, blknode_h, counts2_h, feat_h, out_h, bn, gsrcA, gsrcB,
        rowsA, rowsB, acc, cbuf2, semLA, semLB, semGA, semGB):
    w = _wid()
    neg = jnp.full((16,), -jnp.inf, dtype=jnp.float32)
    arena = w * _ABLK * _BW

    def init(r, _):
      for kk in range(8):
        acc[r, pl.ds(kk * 16, 16)] = neg
      return 0

    lax.fori_loop(0, _ACCR, init, 0)
    pltpu.sync_copy(counts2_h, cbuf2.at[pl.ds(0, 16 * _NW)])
    na = cbuf2[pl.ds(w * 16, 16)][0]
    nbt = na >> 2

    @pl.when(nbt > 0)
    def _():
      def cpn(cc, _):
        pltpu.sync_copy(blknode_h.at[pl.ds(w * _ABLK + cc * 256, 256)],
                        bn.at[pl.ds(cc * 256, 256)])
        return 0

      lax.fori_loop(0, (na + 255) >> 8, cpn, 0)

      slots = ((gsrcA, rowsA, semLA, semGA), (gsrcB, rowsB, semLB, semGB))

      def startL(j, s):
        pltpu.async_copy(blocks_h.at[pl.ds(arena + j * _GB, _GB)],
                         slots[s][0], slots[s][2])

      def startG(s):
        pltpu.async_copy(feat_h.at[slots[s][0]], slots[s][1], slots[s][3])

      def waitL(s):
        pltpu.make_async_copy(blocks_h.at[pl.ds(arena, _GB)], slots[s][0],
                              slots[s][2]).wait()

      def waitG(s):
        pltpu.make_async_copy(feat_h.at[slots[s][0]], slots[s][1],
                              slots[s][3]).wait()

      def accum(j, s):
        rows = slots[s][1]
        nbv = bn[pl.ds(j * 4, 16)]
        for bi in range(4):
          node = nbv[bi]
          ms = tuple(acc[node, pl.ds(kk * 16, 16)] for kk in range(8))

          def red(g, carry):
            out = list(carry)
            for ee in range(8):
              e = bi * 32 + g * 8 + ee
              for kk in range(8):
                out[kk] = jnp.maximum(out[kk], rows[e, pl.ds(kk * 16, 16)])
            return tuple(out)

          ms = lax.fori_loop(0, 4, red, ms)
          for kk in range(8):
            acc[node, pl.ds(kk * 16, 16)] = ms[kk]
        return 0

      startL(0, 0)
      startL(jnp.minimum(1, nbt - 1), 1)
      waitL(0)
      startG(0)

      def ring2(jj, _):
        j0 = jj * 2
        waitL(1)
        startG(1)
        waitG(0)
        startL(jnp.minimum(j0 + 2, nbt - 1), 0)
        accum(j0, 0)
        j1 = jnp.minimum(j0 + 1, nbt - 1)
        waitL(0)
        startG(0)
        waitG(1)
        startL(jnp.minimum(j0 + 3, nbt - 1), 1)
        accum(j1, 1)
        return 0

      lax.fori_loop(0, (nbt + 1) >> 1, ring2, 0)
      waitL(1)
      waitG(0)

    pltpu.sync_copy(acc.at[pl.ds(0, _NPW)], out_h.at[pl.ds(w * _NPW, _NPW)])

  return k


def _seg_max_16():
  @functools.partial(
      pl.kernel,
      mesh=_mesh(),
      out_type=jax.ShapeDtypeStruct((_NPAD * 16,), jnp.float32),
      scratch_types=[
          pltpu.VMEM((_ABLK + 16,), jnp.int32),
          pltpu.VMEM((_GB,), jnp.int32),
          pltpu.VMEM((_GB,), jnp.int32),
          pltpu.VMEM((_GB,), jnp.int32),
          pltpu.VMEM((_GB,), jnp.int32),
          pltpu.VMEM((_GB, 128), jnp.float32),
          pltpu.VMEM((_GB, 128), jnp.float32),
          pltpu.VMEM((_ACCR * 16,), jnp.float32),
          pltpu.VMEM((16 * _NW + 16,), jnp.int32),
          pltpu.SemaphoreType.DMA,
          pltpu.SemaphoreType.DMA,
          pltpu.SemaphoreType.DMA,
          pltpu.SemaphoreType.DMA,
      ],
  )
  def k(blocks_h, blknode_h, counts2_h, hpk_h, out_h, bn, gsrcA, gsrcB,
        gidxA, gidxB, rowsA, rowsB, acc, cbuf2, semLA, semLB, semGA, semGB):
    w = _wid()
    neg = jnp.full((16,), -jnp.inf, dtype=jnp.float32)
    arena = w * _ABLK * _BW

    def init(r, _):
      acc[pl.ds(r * 16, 16)] = neg
      return 0

    lax.fori_loop(0, _ACCR, init, 0)
    pltpu.sync_copy(counts2_h, cbuf2.at[pl.ds(0, 16 * _NW)])
    na = cbuf2[pl.ds(w * 16, 16)][0]
    nbt = na >> 2

    @pl.when(nbt > 0)
    def _():
      def cpn(cc, _):
        pltpu.sync_copy(blknode_h.at[pl.ds(w * _ABLK + cc * 256, 256)],
                        bn.at[pl.ds(cc * 256, 256)])
        return 0

      lax.fori_loop(0, (na + 255) >> 8, cpn, 0)

      slots = ((gsrcA, gidxA, rowsA, semLA, semGA),
               (gsrcB, gidxB, rowsB, semLB, semGB))

      def startL(j, s):
        pltpu.async_copy(blocks_h.at[pl.ds(arena + j * _GB, _GB)],
                         slots[s][0], slots[s][3])

      def startG(s):
        gsrc, gidx = slots[s][0], slots[s][1]
        for u in range(_GB // 16):
          gidx[pl.ds(u * 16, 16)] = gsrc[pl.ds(u * 16, 16)] >> 3
        pltpu.async_copy(hpk_h.at[gidx], slots[s][2], slots[s][4])

      def waitL(s):
        pltpu.make_async_copy(blocks_h.at[pl.ds(arena, _GB)], slots[s][0],
                              slots[s][3]).wait()

      def waitG(s):
        pltpu.make_async_copy(hpk_h.at[slots[s][1]], slots[s][2],
                              slots[s][4]).wait()

      def accum(j, s):
        gsrc, rows = slots[s][0], slots[s][2]
        nbv = bn[pl.ds(j * 4, 16)]
        for bi in range(4):
          node = nbv[bi]
          m0 = acc[pl.ds(node * 16, 16)]

          def red(g, carry):
            srcv = gsrc[pl.ds(bi * 32 + g * 16, 16)]
            for ee in range(16):
              sv = srcv[ee]
              e = bi * 32 + g * 16 + ee
              val = rows[e, pl.ds((sv & 7) * 16, 16)]
              carry = jnp.maximum(carry, val)
            return carry

          m0 = lax.fori_loop(0, 2, red, m0)
          acc[pl.ds(node * 16, 16)] = m0
        return 0

      startL(0, 0)
      startL(jnp.minimum(1, nbt - 1), 1)
      waitL(0)
      startG(0)

      def ring2(jj, _):
        j0 = jj * 2
        waitL(1)
        startG(1)
        waitG(0)
        startL(jnp.minimum(j0 + 2, nbt - 1), 0)
        accum(j0, 0)
        j1 = jnp.minimum(j0 + 1, nbt - 1)
        waitL(0)
        startG(0)
        waitG(1)
        startL(jnp.minimum(j0 + 3, nbt - 1), 1)
        accum(j1, 1)
        return 0

      lax.fori_loop(0, (nbt + 1) >> 1, ring2, 0)
      waitL(1)
      waitG(0)

    pltpu.sync_copy(acc.at[pl.ds(0, _NPW * 16)],
                    out_h.at[pl.ds(w * _NPW * 16, _NPW * 16)])

  return k


def _dense1(x, agg, W1, b1):
  def body(x_ref, a_ref, w_ref, b_ref, o_ref):
    a = a_ref[...]
    a = jnp.where(a == -jnp.inf, 0.0, a)
    rst = x_ref[...] + a
    h = jnp.dot(rst, w_ref[...], preferred_element_type=jnp.float32)
    o_ref[...] = jnp.maximum(h + b_ref[...], 0.0)

  br = 2000
  return pl.pallas_call(
      body,
      grid=(_N // br,),
      in_specs=[
          pl.BlockSpec((br, 128), lambda i: (i, 0)),
          pl.BlockSpec((br, 128), lambda i: (i, 0)),
          pl.BlockSpec((128, 16), lambda i: (0, 0)),
          pl.BlockSpec((1, 16), lambda i: (0, 0)),
      ],
      out_specs=pl.BlockSpec((br, 16), lambda i: (i, 0)),
      out_shape=jax.ShapeDtypeStruct((_N, 16), jnp.float32),
  )(x, agg, W1, b1.reshape(1, 16))


def _dense2(h, agg, W2, b2):
  def body(h_ref, a_ref, w_ref, b_ref, o_ref):
    a = a_ref[...]
    a = jnp.where(a == -jnp.inf, 0.0, a)
    rst = h_ref[...] + a
    z = jnp.dot(rst, w_ref[...], preferred_element_type=jnp.float32)
    z = z + b_ref[...]
    m = jnp.max(z, axis=-1, keepdims=True)
    zm = z - m
    o_ref[...] = zm - jnp.log(jnp.sum(jnp.exp(zm), axis=-1, keepdims=True))

  br = 2000
  return pl.pallas_call(
      body,
      grid=(_N // br,),
      in_specs=[
          pl.BlockSpec((br, 16), lambda i: (i, 0)),
          pl.BlockSpec((br, 16), lambda i: (i, 0)),
          pl.BlockSpec((16, 7), lambda i: (0, 0)),
          pl.BlockSpec((1, 7), lambda i: (0, 0)),
      ],
      out_specs=pl.BlockSpec((br, 7), lambda i: (i, 0)),
      out_shape=jax.ShapeDtypeStruct((_N, 7), jnp.float32),
  )(h, agg, W2, b2.reshape(1, 7))


def kernel(edge_index, features, W1, b1, W2, b2):
  src = edge_index[0]
  dst = edge_index[1]
  lists, counts = _bucketize()(src, dst)
  blocks, blknode, counts2 = _node_blocks()(lists, counts)
  agg1 = _seg_max_128()(blocks, blknode, counts2, features)[:_N]
  h = _dense1(features, agg1, W1, b1)
  hpk = jnp.concatenate(
      [h, jnp.zeros((_NPAD - _N, 16), jnp.float32)]).reshape(_NPAD // 8, 128)
  agg2 = _seg_max_16()(blocks, blknode, counts2, hpk).reshape(_NPAD, 16)[:_N]
  return _dense2(h, agg2, W2, b2)
